# baseline (speedup 1.0000x reference)
import functools

import jax
import jax.numpy as jnp
from jax import lax
from jax.experimental import pallas as pl
from jax.experimental.pallas import tpu as pltpu
from jax.experimental.pallas import tpu_sc as plsc

_B, _NQ, _NK, _DIM = 8, 32, 8192, 768
_ID = 64
_BK = 8192
_NKB = _NK // _BK
_SCALE = _ID ** (-0.5)
_NEG = -3.0e38

_NROW = _B * _NQ
_L = 16
_NW = 32
_RPW = _NROW // _NW


def _tc_scores_body(q_ref, wq_ref, k_ref, wk_ref, out_ref, cand_ref, qh_s,
                    cand_s):
    j = pl.program_id(1)

    @pl.when(j == 0)
    def _():
        qh_s[...] = lax.dot_general(
            q_ref[0], wq_ref[...], (((1,), (0,)), ((), ())),
            preferred_element_type=jnp.float32) * _SCALE
        for jj in range(_NKB, 4):
            cand_s[jj] = jnp.full((_NQ, 4), _NEG, jnp.float32)

    kp = lax.dot_general(
        k_ref[0], wk_ref[...], (((1,), (0,)), ((), ())),
        preferred_element_type=jnp.float32)
    s = lax.dot_general(
        qh_s[...], kp, (((1,), (1,)), ((), ())),
        preferred_element_type=jnp.float32)
    out_ref[0] = s

    m1 = jnp.max(s, axis=-1, keepdims=True)
    s2 = jnp.where(s < m1, s, _NEG)
    m2 = jnp.max(s2, axis=-1, keepdims=True)
    s3 = jnp.where(s2 < m2, s2, _NEG)
    m3 = jnp.max(s3, axis=-1, keepdims=True)
    s4 = jnp.where(s3 < m3, s3, _NEG)
    m4 = jnp.max(s4, axis=-1, keepdims=True)
    c1 = jnp.sum(jnp.where(s == m1, 1.0, 0.0), axis=-1, keepdims=True)
    c2 = jnp.sum(jnp.where(s == m2, 1.0, 0.0), axis=-1, keepdims=True)
    c3 = jnp.sum(jnp.where(s == m3, 1.0, 0.0), axis=-1, keepdims=True)
    v1 = m1
    v2 = jnp.where(c1 >= 2.0, m1, m2)
    v3 = jnp.where(c1 >= 3.0, m1, jnp.where(c1 + c2 >= 3.0, m2, m3))
    v4 = jnp.where(c1 >= 4.0, m1,
         jnp.where(c1 + c2 >= 4.0, m2,
         jnp.where(c1 + c2 + c3 >= 4.0, m3, m4)))
    val4 = jnp.concatenate([v1, v2, v3, v4], axis=-1)
    for jj in range(_NKB):
        @pl.when(j == jj)
        def _(jj=jj):
            cand_s[jj] = val4

    @pl.when(j == _NKB - 1)
    def _():
        cand_ref[0] = jnp.concatenate(
            [cand_s[jj] for jj in range(4)], axis=-1)


def _tc_scores(q, k, Wq, Wk):
    nb = q.shape[0]
    return pl.pallas_call(
        _tc_scores_body,
        grid=(nb, _NKB),
        in_specs=[
            pl.BlockSpec((1, _NQ, _DIM), lambda b, j: (b, 0, 0)),
            pl.BlockSpec((_DIM, _ID), lambda b, j: (0, 0)),
            pl.BlockSpec((1, _BK, _DIM), lambda b, j: (b, j, 0)),
            pl.BlockSpec((_DIM, _ID), lambda b, j: (0, 0)),
        ],
        out_specs=[
            pl.BlockSpec((1, _NQ, _BK), lambda b, j: (b, 0, j)),
            pl.BlockSpec((1, _NQ, 16), lambda b, j: (b, 0, 0)),
        ],
        out_shape=[
            jax.ShapeDtypeStruct((nb, _NQ, _NK), jnp.float32),
            jax.ShapeDtypeStruct((nb, _NQ, 16), jnp.float32),
        ],
        scratch_shapes=[
            pltpu.VMEM((_NQ, _ID), jnp.float32),
            pltpu.VMEM((4, _NQ, 4), jnp.float32),
        ],
    )(q, Wq, k, Wk)


def _bubble4(m, x):
    h1 = jnp.maximum(m[0], x)
    l1 = jnp.minimum(m[0], x)
    h2 = jnp.maximum(m[1], l1)
    l2 = jnp.minimum(m[1], l1)
    h3 = jnp.maximum(m[2], l2)
    l3 = jnp.minimum(m[2], l2)
    h4 = jnp.maximum(m[3], l3)
    return [h1, h2, h3, h4]


def _sc_body(rpw, scores_hbm, cands_hbm, out_hbm, row_v, cand_v):
    wid = lax.axis_index("s") * 2 + lax.axis_index("c")

    def row_loop(i, _carry):
        r = wid * rpw + i
        base_w = r * _NK
        pltpu.sync_copy(cands_hbm.at[pl.ds(r * _L, _L)], cand_v)
        pltpu.sync_copy(scores_hbm.at[pl.ds(base_w, _NK)], row_v)

        cand = cand_v[...]
        s1 = s2 = s3 = s4 = jnp.float32(_NEG)
        for lane in range(_L):
            x = cand[lane]
            h1 = jnp.maximum(s1, x)
            l1 = jnp.minimum(s1, x)
            h2 = jnp.maximum(s2, l1)
            l2 = jnp.minimum(s2, l1)
            h3 = jnp.maximum(s3, l2)
            l3 = jnp.minimum(s3, l2)
            s4 = jnp.maximum(s4, l3)
            s1, s2, s3 = h1, h2, h3
        g1, thr = s1, s4

        def p2(t, acc):
            x = row_v[pl.ds(t * _L, _L)]
            p = jnp.where(x >= thr, jnp.exp(x - g1), 0.0)
            row_v[pl.ds(t * _L, _L)] = p
            return acc + p

        acc = lax.fori_loop(0, _NK // _L, p2, jnp.zeros((_L,), jnp.float32),
                            unroll=16)
        denom = jnp.float32(0.0)
        for lane in range(_L):
            denom = denom + acc[lane]
        invd = jnp.ones((_L,), jnp.float32) / jnp.broadcast_to(denom, (_L,))

        def p3(t, c):
            row_v[pl.ds(t * _L, _L)] = row_v[pl.ds(t * _L, _L)] * invd
            return c

        lax.fori_loop(0, _NK // _L, p3, 0, unroll=16)
        pltpu.sync_copy(row_v, out_hbm.at[pl.ds(base_w, _NK)])
        return 0

    lax.fori_loop(0, rpw, row_loop, 0)


def _sc_topk_softmax(flat_scores, flat_cands):
    nrow = flat_scores.shape[0] // _NK
    mesh = plsc.VectorSubcoreMesh(core_axis_name="c", subcore_axis_name="s")
    fn = functools.partial(
        pl.kernel,
        mesh=mesh,
        out_type=jax.ShapeDtypeStruct((nrow * _NK,), jnp.float32),
        scratch_types=[
            pltpu.VMEM((_NK,), jnp.float32),
            pltpu.VMEM((_L,), jnp.float32),
        ],
    )(functools.partial(_sc_body, nrow // _NW))
    return fn(flat_scores, flat_cands)


@jax.jit
def _run(q, k, Wq, Wk):
    scores, cands = _tc_scores(q, k, Wq, Wk)
    out = _sc_topk_softmax(scores.reshape(_B * _NQ * _NK),
                           cands.reshape(_B * _NQ * _L))
    return out.reshape(_B, 1, _NQ, _NK)


def kernel(q, k, v, Wq, Wk):
    del v
    return _run(q, k, Wq, Wk)


# speedup vs baseline: 1.0704x; 1.0704x over previous
import functools

import jax
import jax.numpy as jnp
from jax import lax
from jax.experimental import pallas as pl
from jax.experimental.pallas import tpu as pltpu
from jax.experimental.pallas import tpu_sc as plsc

_B, _NQ, _NK, _DIM = 8, 32, 8192, 768
_ID = 64
_BK = 4096
_NKB = _NK // _BK
_SCALE = _ID ** (-0.5)
_NEG = -3.0e38

_NROW = _B * _NQ
_L = 16
_NW = 32
_RPW = _NROW // _NW


def _tc_scores_body(q_ref, wq_ref, k_ref, wk_ref, out_ref, cand_ref, qh_s,
                    cand_s):
    j = pl.program_id(1)

    @pl.when(j == 0)
    def _():
        qh_s[...] = lax.dot_general(
            q_ref[0], wq_ref[...], (((1,), (0,)), ((), ())),
            preferred_element_type=jnp.float32) * _SCALE
        for jj in range(_NKB, 4):
            cand_s[jj] = jnp.full((_NQ, 4), _NEG, jnp.float32)

    kp = lax.dot_general(
        k_ref[0], wk_ref[...], (((1,), (0,)), ((), ())),
        preferred_element_type=jnp.float32)
    s = lax.dot_general(
        qh_s[...], kp, (((1,), (1,)), ((), ())),
        preferred_element_type=jnp.float32)
    out_ref[0] = s

    m1 = jnp.max(s, axis=-1, keepdims=True)
    s2 = jnp.where(s < m1, s, _NEG)
    m2 = jnp.max(s2, axis=-1, keepdims=True)
    s3 = jnp.where(s2 < m2, s2, _NEG)
    m3 = jnp.max(s3, axis=-1, keepdims=True)
    s4 = jnp.where(s3 < m3, s3, _NEG)
    m4 = jnp.max(s4, axis=-1, keepdims=True)
    c1 = jnp.sum(jnp.where(s == m1, 1.0, 0.0), axis=-1, keepdims=True)
    c2 = jnp.sum(jnp.where(s == m2, 1.0, 0.0), axis=-1, keepdims=True)
    c3 = jnp.sum(jnp.where(s == m3, 1.0, 0.0), axis=-1, keepdims=True)
    v1 = m1
    v2 = jnp.where(c1 >= 2.0, m1, m2)
    v3 = jnp.where(c1 >= 3.0, m1, jnp.where(c1 + c2 >= 3.0, m2, m3))
    v4 = jnp.where(c1 >= 4.0, m1,
         jnp.where(c1 + c2 >= 4.0, m2,
         jnp.where(c1 + c2 + c3 >= 4.0, m3, m4)))
    val4 = jnp.concatenate([v1, v2, v3, v4], axis=-1)
    for jj in range(_NKB):
        @pl.when(j == jj)
        def _(jj=jj):
            cand_s[jj] = val4

    @pl.when(j == _NKB - 1)
    def _():
        cand_ref[0] = jnp.concatenate(
            [cand_s[jj] for jj in range(4)], axis=-1)


def _tc_scores(q, k, Wq, Wk):
    nb = q.shape[0]
    return pl.pallas_call(
        _tc_scores_body,
        grid=(nb, _NKB),
        in_specs=[
            pl.BlockSpec((1, _NQ, _DIM), lambda b, j: (b, 0, 0)),
            pl.BlockSpec((_DIM, _ID), lambda b, j: (0, 0)),
            pl.BlockSpec((1, _BK, _DIM), lambda b, j: (b, j, 0)),
            pl.BlockSpec((_DIM, _ID), lambda b, j: (0, 0)),
        ],
        out_specs=[
            pl.BlockSpec((1, _NQ, _BK), lambda b, j: (b, 0, j)),
            pl.BlockSpec((1, _NQ, 16), lambda b, j: (b, 0, 0)),
        ],
        out_shape=[
            jax.ShapeDtypeStruct((nb, _NQ, _NK), jnp.float32),
            jax.ShapeDtypeStruct((nb, _NQ, 16), jnp.float32),
        ],
        scratch_shapes=[
            pltpu.VMEM((_NQ, _ID), jnp.float32),
            pltpu.VMEM((4, _NQ, 4), jnp.float32),
        ],
    )(q, Wq, k, Wk)


def _bubble4(m, x):
    h1 = jnp.maximum(m[0], x)
    l1 = jnp.minimum(m[0], x)
    h2 = jnp.maximum(m[1], l1)
    l2 = jnp.minimum(m[1], l1)
    h3 = jnp.maximum(m[2], l2)
    l3 = jnp.minimum(m[2], l2)
    h4 = jnp.maximum(m[3], l3)
    return [h1, h2, h3, h4]


def _sc_row_compute(row_v, cand, g_dummy=None):
    s1 = s2 = s3 = s4 = jnp.float32(_NEG)
    for lane in range(_L):
        x = cand[lane]
        h1 = jnp.maximum(s1, x)
        l1 = jnp.minimum(s1, x)
        h2 = jnp.maximum(s2, l1)
        l2 = jnp.minimum(s2, l1)
        h3 = jnp.maximum(s3, l2)
        l3 = jnp.minimum(s3, l2)
        s4 = jnp.maximum(s4, l3)
        s1, s2, s3 = h1, h2, h3
    g1, thr = s1, s4

    def p2(t, acc):
        x = row_v[pl.ds(t * _L, _L)]
        p = jnp.where(x >= thr, jnp.exp(x - g1), 0.0)
        row_v[pl.ds(t * _L, _L)] = p
        return acc + p

    acc = lax.fori_loop(0, _NK // _L, p2, jnp.zeros((_L,), jnp.float32),
                        unroll=16)
    denom = jnp.float32(0.0)
    for lane in range(_L):
        denom = denom + acc[lane]
    invd = jnp.ones((_L,), jnp.float32) / jnp.broadcast_to(denom, (_L,))

    def p3(t, c):
        row_v[pl.ds(t * _L, _L)] = row_v[pl.ds(t * _L, _L)] * invd
        return c

    lax.fori_loop(0, _NK // _L, p3, 0, unroll=16)


def _sc_body(rpw, scores_hbm, cands_hbm, out_hbm,
             row0, row1, row2, cand_all,
             in0, in1, in2, out0, out1, out2):
    wid = lax.axis_index("s") * 2 + lax.axis_index("c")
    base_r = wid * rpw

    pltpu.sync_copy(cands_hbm.at[pl.ds(base_r * _L, rpw * _L)], cand_all)

    bufs = [row0, row1, row2]
    insems = [in0, in1, in2]
    outsems = [out0, out1, out2]
    h_in = [None, None, None]
    h_out = [None, None, None]

    def start_in(s, i):
        h_in[s] = pltpu.async_copy(
            scores_hbm.at[pl.ds((base_r + i) * _NK, _NK)], bufs[s], insems[s])

    def start_out(s, i):
        h_out[s] = pltpu.async_copy(
            bufs[s], out_hbm.at[pl.ds((base_r + i) * _NK, _NK)], outsems[s])

    start_in(0, 0)
    for i in range(rpw):
        s = i % 3
        if i + 1 < rpw:
            nxt = (i + 1) % 3
            if h_out[nxt] is not None:
                h_out[nxt].wait()
            start_in(nxt, i + 1)
        h_in[s].wait()
        cand = cand_all[pl.ds(i * _L, _L)]
        _sc_row_compute(bufs[s], cand)
        start_out(s, i)
    for s in range(3):
        if h_out[s] is not None:
            h_out[s].wait()


def _sc_topk_softmax(flat_scores, flat_cands):
    nrow = flat_scores.shape[0] // _NK
    rpw = nrow // _NW
    mesh = plsc.VectorSubcoreMesh(core_axis_name="c", subcore_axis_name="s")
    fn = functools.partial(
        pl.kernel,
        mesh=mesh,
        out_type=jax.ShapeDtypeStruct((nrow * _NK,), jnp.float32),
        scratch_types=[
            pltpu.VMEM((_NK,), jnp.float32),
            pltpu.VMEM((_NK,), jnp.float32),
            pltpu.VMEM((_NK,), jnp.float32),
            pltpu.VMEM((rpw * _L,), jnp.float32),
            pltpu.SemaphoreType.DMA,
            pltpu.SemaphoreType.DMA,
            pltpu.SemaphoreType.DMA,
            pltpu.SemaphoreType.DMA,
            pltpu.SemaphoreType.DMA,
            pltpu.SemaphoreType.DMA,
        ],
    )(functools.partial(_sc_body, rpw))
    return fn(flat_scores, flat_cands)


@jax.jit
def _run(q, k, Wq, Wk):
    scores, cands = _tc_scores(q, k, Wq, Wk)
    out = _sc_topk_softmax(scores.reshape(_B * _NQ * _NK),
                           cands.reshape(_B * _NQ * _L))
    return out.reshape(_B, 1, _NQ, _NK)


def kernel(q, k, v, Wq, Wk):
    del v
    return _run(q, k, Wq, Wk)


# speedup vs baseline: 1.1766x; 1.0992x over previous
import functools

import jax
import jax.numpy as jnp
from jax import lax
from jax.experimental import pallas as pl
from jax.experimental.pallas import tpu as pltpu
from jax.experimental.pallas import tpu_sc as plsc

_B, _NQ, _NK, _DIM = 8, 32, 8192, 768
_ID = 64
_BK = 4096
_NKB = _NK // _BK
_SCALE = _ID ** (-0.5)
_NEG = -3.0e38

_NROW = _B * _NQ
_L = 16
_NW = 32
_RPW = _NROW // _NW


def _tc_scores_body(q_ref, wq_ref, k_ref, wk_ref, out_ref, cand_ref, qh_s,
                    cand_s):
    j = pl.program_id(1)

    @pl.when(j == 0)
    def _():
        qh_s[...] = lax.dot_general(
            q_ref[0], wq_ref[...], (((1,), (0,)), ((), ())),
            preferred_element_type=jnp.float32) * _SCALE
        for jj in range(_NKB, 4):
            cand_s[jj] = jnp.full((_NQ, 4), _NEG, jnp.float32)

    kp = lax.dot_general(
        k_ref[0], wk_ref[...], (((1,), (0,)), ((), ())),
        preferred_element_type=jnp.float32)
    s = lax.dot_general(
        qh_s[...], kp, (((1,), (1,)), ((), ())),
        preferred_element_type=jnp.float32)
    out_ref[0] = s

    m1 = jnp.max(s, axis=-1, keepdims=True)
    s2 = jnp.where(s < m1, s, _NEG)
    m2 = jnp.max(s2, axis=-1, keepdims=True)
    s3 = jnp.where(s2 < m2, s2, _NEG)
    m3 = jnp.max(s3, axis=-1, keepdims=True)
    s4 = jnp.where(s3 < m3, s3, _NEG)
    m4 = jnp.max(s4, axis=-1, keepdims=True)
    c1 = jnp.sum(jnp.where(s == m1, 1.0, 0.0), axis=-1, keepdims=True)
    c2 = jnp.sum(jnp.where(s == m2, 1.0, 0.0), axis=-1, keepdims=True)
    c3 = jnp.sum(jnp.where(s == m3, 1.0, 0.0), axis=-1, keepdims=True)
    v1 = m1
    v2 = jnp.where(c1 >= 2.0, m1, m2)
    v3 = jnp.where(c1 >= 3.0, m1, jnp.where(c1 + c2 >= 3.0, m2, m3))
    v4 = jnp.where(c1 >= 4.0, m1,
         jnp.where(c1 + c2 >= 4.0, m2,
         jnp.where(c1 + c2 + c3 >= 4.0, m3, m4)))
    val4 = jnp.concatenate([v1, v2, v3, v4], axis=-1)
    for jj in range(_NKB):
        @pl.when(j == jj)
        def _(jj=jj):
            cand_s[jj] = val4

    @pl.when(j == _NKB - 1)
    def _():
        cand_ref[0] = jnp.concatenate(
            [cand_s[jj] for jj in range(4)], axis=-1)


def _tc_scores(q, k, Wq, Wk):
    nb = q.shape[0]
    return pl.pallas_call(
        _tc_scores_body,
        grid=(nb, _NKB),
        in_specs=[
            pl.BlockSpec((1, _NQ, _DIM), lambda b, j: (b, 0, 0)),
            pl.BlockSpec((_DIM, _ID), lambda b, j: (0, 0)),
            pl.BlockSpec((1, _BK, _DIM), lambda b, j: (b, j, 0)),
            pl.BlockSpec((_DIM, _ID), lambda b, j: (0, 0)),
        ],
        out_specs=[
            pl.BlockSpec((1, _NQ, _BK), lambda b, j: (b, 0, j)),
            pl.BlockSpec((1, _NQ, 16), lambda b, j: (b, 0, 0)),
        ],
        out_shape=[
            jax.ShapeDtypeStruct((nb, _NQ, _NK), jnp.float32),
            jax.ShapeDtypeStruct((nb, _NQ, 16), jnp.float32),
        ],
        scratch_shapes=[
            pltpu.VMEM((_NQ, _ID), jnp.float32),
            pltpu.VMEM((4, _NQ, 4), jnp.float32),
        ],
    )(q, Wq, k, Wk)


def _bubble4(m, x):
    h1 = jnp.maximum(m[0], x)
    l1 = jnp.minimum(m[0], x)
    h2 = jnp.maximum(m[1], l1)
    l2 = jnp.minimum(m[1], l1)
    h3 = jnp.maximum(m[2], l2)
    l3 = jnp.minimum(m[2], l2)
    h4 = jnp.maximum(m[3], l3)
    return [h1, h2, h3, h4]


def _sc_row_compute(row_v, cand, g_dummy=None):
    s1 = s2 = s3 = s4 = jnp.float32(_NEG)
    for lane in range(_L):
        x = cand[lane]
        h1 = jnp.maximum(s1, x)
        l1 = jnp.minimum(s1, x)
        h2 = jnp.maximum(s2, l1)
        l2 = jnp.minimum(s2, l1)
        h3 = jnp.maximum(s3, l2)
        l3 = jnp.minimum(s3, l2)
        s4 = jnp.maximum(s4, l3)
        s1, s2, s3 = h1, h2, h3
    g1, thr = s1, s4

    def p2(t, acc):
        x = row_v[pl.ds(t * _L, _L)]
        p = jnp.where(x >= thr, jnp.exp(x - g1), 0.0)
        row_v[pl.ds(t * _L, _L)] = p
        return acc + p

    acc = lax.fori_loop(0, _NK // _L, p2, jnp.zeros((_L,), jnp.float32),
                        unroll=16)
    denom = jnp.float32(0.0)
    for lane in range(_L):
        denom = denom + acc[lane]
    invd = jnp.ones((_L,), jnp.float32) / jnp.broadcast_to(denom, (_L,))

    def p3(t, c):
        row_v[pl.ds(t * _L, _L)] = row_v[pl.ds(t * _L, _L)] * invd
        return c

    lax.fori_loop(0, _NK // _L, p3, 0, unroll=16)


def _sc_body(rpw, scores_hbm, cands_hbm, out_hbm,
             row0, row1, row2, cand_all,
             in0, in1, in2, out0, out1, out2):
    wid = lax.axis_index("s") * 2 + lax.axis_index("c")
    base_r = wid * rpw

    pltpu.sync_copy(cands_hbm.at[pl.ds(base_r * _L, rpw * _L)], cand_all)

    bufs = [row0, row1, row2]
    insems = [in0, in1, in2]
    outsems = [out0, out1, out2]
    h_in = [None, None, None]
    h_out = [None, None, None]

    def start_in(s, i):
        r = base_r + i
        h_in[s] = pltpu.async_copy(
            scores_hbm.at[r // _NQ, r % _NQ], bufs[s], insems[s])

    def start_out(s, i):
        h_out[s] = pltpu.async_copy(
            bufs[s], out_hbm.at[pl.ds((base_r + i) * _NK, _NK)], outsems[s])

    start_in(0, 0)
    for i in range(rpw):
        s = i % 3
        if i + 1 < rpw:
            nxt = (i + 1) % 3
            if h_out[nxt] is not None:
                h_out[nxt].wait()
            start_in(nxt, i + 1)
        h_in[s].wait()
        cand = cand_all[pl.ds(i * _L, _L)]
        _sc_row_compute(bufs[s], cand)
        start_out(s, i)
    for s in range(3):
        if h_out[s] is not None:
            h_out[s].wait()


def _sc_topk_softmax(scores, flat_cands):
    nrow = scores.shape[0] * scores.shape[1]
    rpw = nrow // _NW
    mesh = plsc.VectorSubcoreMesh(core_axis_name="c", subcore_axis_name="s")
    fn = functools.partial(
        pl.kernel,
        mesh=mesh,
        out_type=jax.ShapeDtypeStruct((nrow * _NK,), jnp.float32),
        scratch_types=[
            pltpu.VMEM((_NK,), jnp.float32),
            pltpu.VMEM((_NK,), jnp.float32),
            pltpu.VMEM((_NK,), jnp.float32),
            pltpu.VMEM((rpw * _L,), jnp.float32),
            pltpu.SemaphoreType.DMA,
            pltpu.SemaphoreType.DMA,
            pltpu.SemaphoreType.DMA,
            pltpu.SemaphoreType.DMA,
            pltpu.SemaphoreType.DMA,
            pltpu.SemaphoreType.DMA,
        ],
    )(functools.partial(_sc_body, rpw))
    return fn(scores, flat_cands)


@jax.jit
def _run(q, k, Wq, Wk):
    scores, cands = _tc_scores(q, k, Wq, Wk)
    out = _sc_topk_softmax(scores, cands.reshape(_B * _NQ * _L))
    return out.reshape(_B, 1, _NQ, _NK)


def kernel(q, k, v, Wq, Wk):
    del v
    return _run(q, k, Wq, Wk)


# speedup vs baseline: 1.3033x; 1.1077x over previous
import functools

import jax
import jax.numpy as jnp
from jax import lax
from jax.experimental import pallas as pl
from jax.experimental.pallas import tpu as pltpu
from jax.experimental.pallas import tpu_sc as plsc

_B, _NQ, _NK, _DIM = 8, 32, 8192, 768
_ID = 64
_BK = 4096
_NKB = _NK // _BK
_SCALE = _ID ** (-0.5)
_NEG = -3.0e38

_NROW = _B * _NQ
_L = 16
_NW = 32
_RPW = _NROW // _NW


def _tc_scores_body(q_ref, wq_ref, k_ref, wk_ref, out_ref, cand_ref, qh_s,
                    cand_s):
    j = pl.program_id(1)

    @pl.when(j == 0)
    def _():
        qh_s[...] = lax.dot_general(
            q_ref[0], wq_ref[...], (((1,), (0,)), ((), ())),
            preferred_element_type=jnp.float32) * _SCALE
        for jj in range(_NKB, 4):
            cand_s[jj] = jnp.full((_NQ, 4), _NEG, jnp.float32)

    kp = lax.dot_general(
        k_ref[0], wk_ref[...], (((1,), (0,)), ((), ())),
        preferred_element_type=jnp.float32)
    s = lax.dot_general(
        qh_s[...], kp, (((1,), (1,)), ((), ())),
        preferred_element_type=jnp.float32)
    out_ref[0] = s

    m1 = jnp.max(s, axis=-1, keepdims=True)
    s2 = jnp.where(s < m1, s, _NEG)
    m2 = jnp.max(s2, axis=-1, keepdims=True)
    s3 = jnp.where(s2 < m2, s2, _NEG)
    m3 = jnp.max(s3, axis=-1, keepdims=True)
    s4 = jnp.where(s3 < m3, s3, _NEG)
    m4 = jnp.max(s4, axis=-1, keepdims=True)
    c1 = jnp.sum(jnp.where(s == m1, 1.0, 0.0), axis=-1, keepdims=True)
    c2 = jnp.sum(jnp.where(s == m2, 1.0, 0.0), axis=-1, keepdims=True)
    c3 = jnp.sum(jnp.where(s == m3, 1.0, 0.0), axis=-1, keepdims=True)
    v1 = m1
    v2 = jnp.where(c1 >= 2.0, m1, m2)
    v3 = jnp.where(c1 >= 3.0, m1, jnp.where(c1 + c2 >= 3.0, m2, m3))
    v4 = jnp.where(c1 >= 4.0, m1,
         jnp.where(c1 + c2 >= 4.0, m2,
         jnp.where(c1 + c2 + c3 >= 4.0, m3, m4)))
    val4 = jnp.concatenate([v1, v2, v3, v4], axis=-1)
    for jj in range(_NKB):
        @pl.when(j == jj)
        def _(jj=jj):
            cand_s[jj] = val4

    @pl.when(j == _NKB - 1)
    def _():
        cand_ref[0] = jnp.concatenate(
            [cand_s[jj] for jj in range(4)], axis=-1)


def _tc_scores(q, k, Wq, Wk):
    nb = q.shape[0]
    return pl.pallas_call(
        _tc_scores_body,
        grid=(nb, _NKB),
        in_specs=[
            pl.BlockSpec((1, _NQ, _DIM), lambda b, j: (b, 0, 0)),
            pl.BlockSpec((_DIM, _ID), lambda b, j: (0, 0)),
            pl.BlockSpec((1, _BK, _DIM), lambda b, j: (b, j, 0)),
            pl.BlockSpec((_DIM, _ID), lambda b, j: (0, 0)),
        ],
        out_specs=[
            pl.BlockSpec((1, _NQ, _BK), lambda b, j: (b, 0, j)),
            pl.BlockSpec((1, _NQ, 16), lambda b, j: (b, 0, 0)),
        ],
        out_shape=[
            jax.ShapeDtypeStruct((nb, _NQ, _NK), jnp.float32),
            jax.ShapeDtypeStruct((nb, _NQ, 16), jnp.float32),
        ],
        scratch_shapes=[
            pltpu.VMEM((_NQ, _ID), jnp.float32),
            pltpu.VMEM((4, _NQ, 4), jnp.float32),
        ],
    )(q, Wq, k, Wk)


def _bubble4(m, x):
    h1 = jnp.maximum(m[0], x)
    l1 = jnp.minimum(m[0], x)
    h2 = jnp.maximum(m[1], l1)
    l2 = jnp.minimum(m[1], l1)
    h3 = jnp.maximum(m[2], l2)
    l3 = jnp.minimum(m[2], l2)
    h4 = jnp.maximum(m[3], l3)
    return [h1, h2, h3, h4]


def _sc_row_compute(row_v, cand, g_dummy=None):
    s1 = s2 = s3 = s4 = jnp.float32(_NEG)
    for lane in range(_L):
        x = cand[lane]
        h1 = jnp.maximum(s1, x)
        l1 = jnp.minimum(s1, x)
        h2 = jnp.maximum(s2, l1)
        l2 = jnp.minimum(s2, l1)
        h3 = jnp.maximum(s3, l2)
        l3 = jnp.minimum(s3, l2)
        s4 = jnp.maximum(s4, l3)
        s1, s2, s3 = h1, h2, h3
    g1, thr = s1, s4

    def p2(t, acc):
        x = row_v[pl.ds(t * _L, _L)]
        p = jnp.where(x >= thr, jnp.exp(x - g1), 0.0)
        row_v[pl.ds(t * _L, _L)] = p
        return acc + p

    acc = lax.fori_loop(0, _NK // _L, p2, jnp.zeros((_L,), jnp.float32),
                        unroll=16)
    denom = jnp.float32(0.0)
    for lane in range(_L):
        denom = denom + acc[lane]
    invd = jnp.ones((_L,), jnp.float32) / jnp.broadcast_to(denom, (_L,))

    def p3(t, c):
        row_v[pl.ds(t * _L, _L)] = row_v[pl.ds(t * _L, _L)] * invd
        return c

    lax.fori_loop(0, _NK // _L, p3, 0, unroll=16)


def _sc_body(rpw, scores_hbm, cands_hbm, out_hbm,
             row0, row1, row2, cand_all,
             in0, in1, in2, out0, out1, out2):
    wid = lax.axis_index("s") * 2 + lax.axis_index("c")
    base_r = wid * rpw

    pltpu.sync_copy(
        cands_hbm.at[base_r // _NQ, pl.ds(base_r % _NQ, rpw)], cand_all)

    bufs = [row0, row1, row2]
    insems = [in0, in1, in2]
    outsems = [out0, out1, out2]
    h_in = [None, None, None]
    h_out = [None, None, None]

    def start_in(s, i):
        r = base_r + i
        h_in[s] = pltpu.async_copy(
            scores_hbm.at[r // _NQ, r % _NQ], bufs[s], insems[s])

    def start_out(s, i):
        r = base_r + i
        h_out[s] = pltpu.async_copy(
            bufs[s], out_hbm.at[r // _NQ, 0, r % _NQ], outsems[s])

    start_in(0, 0)
    for i in range(rpw):
        s = i % 3
        if i + 1 < rpw:
            nxt = (i + 1) % 3
            if h_out[nxt] is not None:
                h_out[nxt].wait()
            start_in(nxt, i + 1)
        h_in[s].wait()
        cand = cand_all.at[i][...]
        _sc_row_compute(bufs[s], cand)
        start_out(s, i)
    for s in range(3):
        if h_out[s] is not None:
            h_out[s].wait()


def _sc_topk_softmax(scores, cands):
    nb = scores.shape[0]
    nrow = nb * scores.shape[1]
    rpw = nrow // _NW
    mesh = plsc.VectorSubcoreMesh(core_axis_name="c", subcore_axis_name="s")
    fn = functools.partial(
        pl.kernel,
        mesh=mesh,
        out_type=jax.ShapeDtypeStruct((nb, 1, _NQ, _NK), jnp.float32),
        scratch_types=[
            pltpu.VMEM((_NK,), jnp.float32),
            pltpu.VMEM((_NK,), jnp.float32),
            pltpu.VMEM((_NK,), jnp.float32),
            pltpu.VMEM((rpw, _L), jnp.float32),
            pltpu.SemaphoreType.DMA,
            pltpu.SemaphoreType.DMA,
            pltpu.SemaphoreType.DMA,
            pltpu.SemaphoreType.DMA,
            pltpu.SemaphoreType.DMA,
            pltpu.SemaphoreType.DMA,
        ],
    )(functools.partial(_sc_body, rpw))
    return fn(scores, cands)


@jax.jit
def _run(q, k, Wq, Wk):
    scores, cands = _tc_scores(q, k, Wq, Wk)
    return _sc_topk_softmax(scores, cands)


def kernel(q, k, v, Wq, Wk):
    del v
    return _run(q, k, Wq, Wk)


# speedup vs baseline: 1.3069x; 1.0028x over previous
import functools

import jax
import jax.numpy as jnp
from jax import lax
from jax.experimental import pallas as pl
from jax.experimental.pallas import tpu as pltpu
from jax.experimental.pallas import tpu_sc as plsc

_B, _NQ, _NK, _DIM = 8, 32, 8192, 768
_ID = 64
_BK = 4096
_NKB = _NK // _BK
_SCALE = _ID ** (-0.5)
_NEG = -3.0e38

_L = 16
_NW = 32


def _tc_scores_body(q_ref, wq_ref, k_ref, wk_ref, out_ref, cand_ref, qh_s,
                    cand_s):
    j = pl.program_id(1)

    @pl.when(j == 0)
    def _():
        qh_s[...] = lax.dot_general(
            q_ref[0], wq_ref[...], (((1,), (0,)), ((), ())),
            preferred_element_type=jnp.float32) * _SCALE
        for jj in range(_NKB, 4):
            cand_s[jj] = jnp.full((_NQ, 4), _NEG, jnp.float32)

    kp = lax.dot_general(
        k_ref[0], wk_ref[...], (((1,), (0,)), ((), ())),
        preferred_element_type=jnp.float32)
    s = lax.dot_general(
        qh_s[...], kp, (((1,), (1,)), ((), ())),
        preferred_element_type=jnp.float32)
    out_ref[0] = s

    m1 = jnp.max(s, axis=-1, keepdims=True)
    s2 = jnp.where(s < m1, s, _NEG)
    m2 = jnp.max(s2, axis=-1, keepdims=True)
    s3 = jnp.where(s2 < m2, s2, _NEG)
    m3 = jnp.max(s3, axis=-1, keepdims=True)
    s4 = jnp.where(s3 < m3, s3, _NEG)
    m4 = jnp.max(s4, axis=-1, keepdims=True)
    c1 = jnp.sum(jnp.where(s == m1, 1.0, 0.0), axis=-1, keepdims=True)
    c2 = jnp.sum(jnp.where(s == m2, 1.0, 0.0), axis=-1, keepdims=True)
    c3 = jnp.sum(jnp.where(s == m3, 1.0, 0.0), axis=-1, keepdims=True)
    v1 = m1
    v2 = jnp.where(c1 >= 2.0, m1, m2)
    v3 = jnp.where(c1 >= 3.0, m1, jnp.where(c1 + c2 >= 3.0, m2, m3))
    v4 = jnp.where(c1 >= 4.0, m1,
         jnp.where(c1 + c2 >= 4.0, m2,
         jnp.where(c1 + c2 + c3 >= 4.0, m3, m4)))
    val4 = jnp.concatenate([v1, v2, v3, v4], axis=-1)
    for jj in range(_NKB):
        @pl.when(j == jj)
        def _(jj=jj):
            cand_s[jj] = val4

    @pl.when(j == _NKB - 1)
    def _():
        cand_ref[0] = jnp.concatenate(
            [cand_s[jj] for jj in range(4)], axis=-1)


def _tc_scores(q, k, Wq, Wk):
    nb = q.shape[0]
    return pl.pallas_call(
        _tc_scores_body,
        grid=(nb, _NKB),
        in_specs=[
            pl.BlockSpec((1, _NQ, _DIM), lambda b, j: (b, 0, 0)),
            pl.BlockSpec((_DIM, _ID), lambda b, j: (0, 0)),
            pl.BlockSpec((1, _BK, _DIM), lambda b, j: (b, j, 0)),
            pl.BlockSpec((_DIM, _ID), lambda b, j: (0, 0)),
        ],
        out_specs=[
            pl.BlockSpec((1, _NQ, _BK), lambda b, j: (b, 0, j)),
            pl.BlockSpec((1, _NQ, 16), lambda b, j: (b, 0, 0)),
        ],
        out_shape=[
            jax.ShapeDtypeStruct((nb, _NQ, _NK), jnp.float32),
            jax.ShapeDtypeStruct((nb, _NQ, 16), jnp.float32),
        ],
        scratch_shapes=[
            pltpu.VMEM((_NQ, _ID), jnp.float32),
            pltpu.VMEM((4, _NQ, 4), jnp.float32),
        ],
    )(q, Wq, k, Wk)


def _sc_row_compute(row_v, cand):
    s1 = s2 = s3 = s4 = jnp.float32(_NEG)
    for lane in range(_L):
        x = cand[lane]
        h1 = jnp.maximum(s1, x)
        l1 = jnp.minimum(s1, x)
        h2 = jnp.maximum(s2, l1)
        l2 = jnp.minimum(s2, l1)
        h3 = jnp.maximum(s3, l2)
        l3 = jnp.minimum(s3, l2)
        s4 = jnp.maximum(s4, l3)
        s1, s2, s3 = h1, h2, h3
    g1, thr = s1, s4

    def p2(t, acc):
        x = row_v[pl.ds(t * _L, _L)]
        p = jnp.where(x >= thr, jnp.exp(x - g1), 0.0)
        row_v[pl.ds(t * _L, _L)] = p
        return acc + p

    acc = lax.fori_loop(0, _NK // _L, p2, jnp.zeros((_L,), jnp.float32),
                        unroll=16)
    denom = jnp.float32(0.0)
    for lane in range(_L):
        denom = denom + acc[lane]
    invd = jnp.ones((_L,), jnp.float32) / jnp.broadcast_to(denom, (_L,))

    def p3(t, c):
        row_v[pl.ds(t * _L, _L)] = row_v[pl.ds(t * _L, _L)] * invd
        return c

    lax.fori_loop(0, _NK // _L, p3, 0, unroll=16)


def _sc_body(rpw, scores_hbm, cands_hbm, out_hbm,
             row0, row1, row2, cand_all,
             in0, in1, in2, out0, out1, out2):
    wid = lax.axis_index("s") * 2 + lax.axis_index("c")
    base_r = wid * rpw

    pltpu.sync_copy(
        cands_hbm.at[base_r // _NQ, pl.ds(base_r % _NQ, rpw)], cand_all)

    bufs = [row0, row1, row2]
    insems = [in0, in1, in2]
    outsems = [out0, out1, out2]
    h_in = [None, None, None]
    h_out = [None, None, None]

    def start_in(s, i):
        r = base_r + i
        h_in[s] = pltpu.async_copy(
            scores_hbm.at[r // _NQ, r % _NQ], bufs[s], insems[s])

    def start_out(s, i):
        r = base_r + i
        h_out[s] = pltpu.async_copy(
            bufs[s], out_hbm.at[r // _NQ, 0, r % _NQ], outsems[s])

    start_in(0, 0)
    for i in range(rpw):
        s = i % 3
        if i + 1 < rpw:
            nxt = (i + 1) % 3
            if h_out[nxt] is not None:
                h_out[nxt].wait()
            start_in(nxt, i + 1)
        h_in[s].wait()
        cand = cand_all.at[i][...]
        _sc_row_compute(bufs[s], cand)
        start_out(s, i)
    for s in range(3):
        if h_out[s] is not None:
            h_out[s].wait()


def _sc_topk_softmax(scores, cands):
    nb = scores.shape[0]
    nrow = nb * scores.shape[1]
    rpw = nrow // _NW
    mesh = plsc.VectorSubcoreMesh(core_axis_name="c", subcore_axis_name="s")
    fn = functools.partial(
        pl.kernel,
        mesh=mesh,
        out_type=jax.ShapeDtypeStruct((nb, 1, _NQ, _NK), jnp.float32),
        scratch_types=[
            pltpu.VMEM((_NK,), jnp.float32),
            pltpu.VMEM((_NK,), jnp.float32),
            pltpu.VMEM((_NK,), jnp.float32),
            pltpu.VMEM((rpw, _L), jnp.float32),
            pltpu.SemaphoreType.DMA,
            pltpu.SemaphoreType.DMA,
            pltpu.SemaphoreType.DMA,
            pltpu.SemaphoreType.DMA,
            pltpu.SemaphoreType.DMA,
            pltpu.SemaphoreType.DMA,
        ],
    )(functools.partial(_sc_body, rpw))
    return fn(scores, cands)


@jax.jit
def _run(q, k, Wq, Wk):
    scores, cands = _tc_scores(q, k, Wq, Wk)
    return _sc_topk_softmax(scores, cands)


def kernel(q, k, v, Wq, Wk):
    del v
    return _run(q, k, Wq, Wk)
